# Initial kernel scaffold; baseline (speedup 1.0000x reference)
import jax
import jax.numpy as jnp
from jax.experimental import pallas as pl


def kernel(anchors, deltas, scores):
    raise NotImplementedError("write your pallas kernel here")


# speedup vs baseline: 15.7002x; 15.7002x over previous
import functools
import math

import jax
import jax.numpy as jnp
from jax import lax
from jax.experimental import pallas as pl
from jax.experimental.pallas import tpu as pltpu

_N = 20000
_R = 20
_C = 1024
_NP = _R * _C
_K1 = 1000
_K2 = 300
_THRESH = 0.7
_IMG = 1024.0
_SCALE_CLAMP = math.log(1000.0 / 16.0)


def _shift_lanes(y, d):
    return jnp.concatenate([jnp.zeros((y.shape[0], d), y.dtype), y[:, : y.shape[1] - d]], axis=1)


def _shift_rows(z, d):
    return jnp.concatenate([jnp.zeros((d, z.shape[1]), z.dtype), z[: z.shape[0] - d]], axis=0)


def _excl_cumsum_flat(x):
    y = x
    d = 1
    while d < _C:
        y = y + _shift_lanes(y, d)
        d *= 2
    row_tot = y[:, _C - 1 : _C]
    z = row_tot
    d = 1
    while d < _R:
        z = z + _shift_rows(z, d)
        d *= 2
    offs = z - row_tot
    return y - x + offs


def _body(ax1_ref, ay1_ref, ax2_ref, ay2_ref, dx_ref, dy_ref, dw_ref, dh_ref,
          sc_ref, out_ref, vals_ref):
    f32 = jnp.float32
    i32 = jnp.int32

    ax1 = ax1_ref[...]
    ay1 = ay1_ref[...]
    ax2 = ax2_ref[...]
    ay2 = ay2_ref[...]
    dx = dx_ref[...]
    dy = dy_ref[...]
    dw = dw_ref[...]
    dh = dh_ref[...]
    sc = sc_ref[...]

    widths = ax2 - ax1
    heights = ay2 - ay1
    ctr_x = ax1 + 0.5 * widths
    ctr_y = ay1 + 0.5 * heights
    dwc = jnp.minimum(dw, _SCALE_CLAMP)
    dhc = jnp.minimum(dh, _SCALE_CLAMP)
    pred_cx = dx * widths + ctr_x
    pred_cy = dy * heights + ctr_y
    pred_w = jnp.exp(dwc) * widths
    pred_h = jnp.exp(dhc) * heights
    x1 = jnp.clip(pred_cx - 0.5 * pred_w, 0.0, _IMG)
    y1 = jnp.clip(pred_cy - 0.5 * pred_h, 0.0, _IMG)
    x2 = jnp.clip(pred_cx + 0.5 * pred_w, 0.0, _IMG)
    y2 = jnp.clip(pred_cy + 0.5 * pred_h, 0.0, _IMG)
    valid = jnp.logical_and(x2 - x1 >= 0.0, y2 - y1 >= 0.0)
    ms = jnp.where(valid, sc, -jnp.inf)

    ibits = lax.bitcast_convert_type(ms, i32)
    key = jnp.where(ibits < 0, ibits ^ jnp.int32(0x7FFFFFFF), ibits)

    def bs_body(_, carry):
        lo, hi = carry
        x = lo ^ hi
        mid = (lo & hi) + (x >> 1) + (x & 1)
        cnt = jnp.sum((key >= mid).astype(i32))
        ge = cnt >= _K1
        return jnp.where(ge, mid, lo), jnp.where(ge, hi, mid - 1)

    t_key, _ = lax.fori_loop(
        0, 34, bs_body, (jnp.int32(-(2**31)), jnp.int32(2**31 - 1)))

    sel_gt = key > t_key
    sel_eq = key == t_key
    count_gt = jnp.sum(sel_gt.astype(i32))
    extra = _K1 - count_gt
    csum_eq = _excl_cumsum_flat(sel_eq.astype(i32))
    sel = jnp.logical_or(sel_gt, jnp.logical_and(sel_eq, csum_eq < extra))
    slot = _excl_cumsum_flat(sel.astype(i32))

    vals_ref[:, 0, :] = x1
    vals_ref[:, 1, :] = y1
    vals_ref[:, 2, :] = x2
    vals_ref[:, 3, :] = y2
    vals_ref[:, 4, :] = ms
    vals_ref[:, 5, :] = slot.astype(f32)
    vals_ref[:, 6, :] = jnp.where(sel, 1.0, 0.0)
    vals_ref[:, 7, :] = jnp.zeros((_R, _C), f32)

    cols_k1 = lax.broadcasted_iota(i32, (1, _K1), 1).astype(f32)
    neg_inf = jnp.float32(-jnp.inf)

    def chunk_body(r, packed):
        v = vals_ref[pl.ds(r, 1), :, :][0]
        pm = jnp.logical_and(v[5][:, None] == cols_k1,
                             v[6][:, None] > 0.0)
        out = []
        for ch in range(5):
            cand = jnp.where(pm, v[ch][:, None], neg_inf)
            out.append(jnp.maximum(packed[ch], jnp.max(cand, axis=0)))
        return tuple(out)

    init = tuple(jnp.full((_K1,), neg_inf, f32) for _ in range(5))
    px1, py1, px2, py2, psc = lax.fori_loop(0, _R, chunk_body, init)

    p_iota = lax.broadcasted_iota(i32, (_K1, _K1), 0)
    q_iota = lax.broadcasted_iota(i32, (_K1, _K1), 1)
    qlt = q_iota < p_iota
    sp = psc[:, None]
    sq = psc[None, :]
    rank = jnp.sum(
        jnp.logical_or(sq > sp, jnp.logical_and(sq == sp, qlt)).astype(i32),
        axis=1)
    cols_k1i = lax.broadcasted_iota(i32, (1, _K1), 1)
    pmk = rank[:, None] == cols_k1i

    def permute(v, fill=neg_inf):
        return jnp.max(jnp.where(pmk, v[:, None], fill), axis=0)

    sx1 = permute(px1)
    sy1 = permute(py1)
    sx2 = permute(px2)
    sy2 = permute(py2)
    ss = permute(psc)

    area = (sx2 - sx1) * (sy2 - sy1)
    iw = jnp.clip(jnp.minimum(sx2[:, None], sx2[None, :]) -
                  jnp.maximum(sx1[:, None], sx1[None, :]), 0.0, None)
    ih = jnp.clip(jnp.minimum(sy2[:, None], sy2[None, :]) -
                  jnp.maximum(sy1[:, None], sy1[None, :]), 0.0, None)
    inter = iw * ih
    union = jnp.maximum(area[:, None] + area[None, :] - inter, 1e-6)
    iou = inter / union
    mf = jnp.where(
        jnp.logical_and(iou > _THRESH, p_iota < q_iota), 1.0, 0.0)

    def fp_cond(carry):
        return carry[1] > 0

    def fp_body(carry):
        alive, _ = carry
        suppressed = jnp.max(mf * alive[:, None], axis=0)
        new = 1.0 - suppressed
        changed = jnp.any(new != alive).astype(i32)
        return new, changed

    alive, _ = lax.while_loop(
        fp_cond, fp_body, (jnp.ones((_K1,), f32), jnp.int32(1)))

    ks = jnp.where(alive > 0.0, ss, neg_inf)
    kp = ks[:, None]
    kq = ks[None, :]
    rank2 = jnp.sum(
        jnp.logical_or(kq > kp, jnp.logical_and(kq == kp, qlt)).astype(i32),
        axis=1)
    cols_k2 = lax.broadcasted_iota(i32, (1, _K2), 1)
    fm = rank2[:, None] == cols_k2

    def pick(v, fill=neg_inf):
        return jnp.max(jnp.where(fm, v[:, None], fill), axis=0)

    ox1 = pick(sx1)
    oy1 = pick(sy1)
    ox2 = pick(sx2)
    oy2 = pick(sy2)
    os_ = pick(ss)
    oalive = pick(alive, 0.0)
    oscore = jnp.where(oalive > 0.0, os_, neg_inf)
    zeros = jnp.zeros((_K2,), f32)
    out_ref[...] = jnp.stack([ox1, oy1, ox2, oy2, oscore, zeros, zeros, zeros])


@jax.jit
def kernel(anchors, deltas, scores):
    pad = _NP - _N

    def chunkify(v, fill=0.0):
        return jnp.pad(v, (0, pad), constant_values=fill).reshape(_R, _C)

    ins = [chunkify(anchors[:, i]) for i in range(4)]
    ins += [chunkify(deltas[:, i]) for i in range(4)]
    ins.append(chunkify(scores, fill=-jnp.inf))

    out = pl.pallas_call(
        _body,
        out_shape=jax.ShapeDtypeStruct((8, _K2), jnp.float32),
        scratch_shapes=[pltpu.VMEM((_R, 8, _C), jnp.float32)],
    )(*ins)

    final_boxes = out[0:4, :].T
    final_scores = out[4, :]
    return final_boxes, final_scores
